# Initial kernel scaffold; baseline (speedup 1.0000x reference)
import jax
import jax.numpy as jnp
from jax.experimental import pallas as pl


def kernel(x, edge_index, W1, b1, W2, b2):
    raise NotImplementedError("write your pallas kernel here")


# speedup vs baseline: 14.5799x; 14.5799x over previous
import functools

import jax
import jax.numpy as jnp
from jax import lax
from jax.experimental import pallas as pl
from jax.experimental.pallas import tpu as pltpu
from jax.experimental.pallas import tpu_sc as plsc

N_NODES = 10000
N_EDGES = 320000
IN_DIM = 128
HID_DIM = 64
OUT_DIM = 64

NC = 2
NS = 16
NW = NC * NS
NPAD = 10240
STRIP = NPAD // NS
EPT = N_EDGES // NW
K = 80
NCHUNK = EPT // K
ROW_BLK = 512
GRID = NPAD // ROW_BLK

_SC_MESH = plsc.VectorSubcoreMesh(
    core_axis_name="c", subcore_axis_name="s", num_cores=NC, num_subcores=NS)


def _zero_rows(rows_v, nrows, width16):
    def zr(r, _):
        def zj(j, c):
            rows_v[r, pl.ds(j * 16, 16)] = jnp.zeros((16,), jnp.float32)
            return c
        return lax.fori_loop(0, width16, zj, _)
    lax.fori_loop(0, nrows, zr, 0)


def _hist_body(col_hbm, out_hbm, idx_c, ones_v, zb_v, deg_sh):
    cid = lax.axis_index("c")
    sid = lax.axis_index("s")
    tid = cid * NS + sid

    def zb(j, c):
        zb_v[pl.ds(j * 16, 16)] = jnp.zeros((16,), jnp.float32)
        return c
    lax.fori_loop(0, STRIP // 16, zb, 0)

    def ob(j, c):
        ones_v[pl.ds(j * 16, 16)] = jnp.ones((16,), jnp.float32)
        return c
    lax.fori_loop(0, K // 16, ob, 0)

    pltpu.sync_copy(zb_v, deg_sh.at[pl.ds(sid * STRIP, STRIP)])
    plsc.subcore_barrier()

    base = tid * EPT

    def step(c, carry):
        off = pl.multiple_of(base + c * K, 8)
        pltpu.sync_copy(col_hbm.at[pl.ds(off, K)], idx_c)
        pltpu.sync_copy(ones_v, deg_sh.at[idx_c], add=True)
        return carry
    lax.fori_loop(0, NCHUNK, step, 0)

    plsc.subcore_barrier()
    pltpu.sync_copy(deg_sh.at[pl.ds(sid * STRIP, STRIP)],
                    out_hbm.at[cid, pl.ds(sid * STRIP, STRIP)])


_hist_call = pl.kernel(
    _hist_body,
    out_type=jax.ShapeDtypeStruct((NC, NPAD), jnp.float32),
    mesh=_SC_MESH,
    scratch_types=[
        pltpu.VMEM((K,), jnp.int32),
        pltpu.VMEM((K,), jnp.float32),
        pltpu.VMEM((STRIP,), jnp.float32),
        pltpu.VMEM_SHARED((NPAD,), jnp.float32),
    ],
)


def _agg_body(s_hbm, row_hbm, col_hbm, out_hbm, idx_r, idx_c, rows_v, acc_sh,
              sem):
    cid = lax.axis_index("c")
    sid = lax.axis_index("s")
    tid = cid * NS + sid

    _zero_rows(rows_v, K, HID_DIM // 16)

    def zs(k, c):
        pltpu.sync_copy(rows_v, acc_sh.at[pl.ds(sid * STRIP + k * K, K)])
        return c
    lax.fori_loop(0, STRIP // K, zs, 0)
    plsc.subcore_barrier()

    base = tid * EPT

    def step(c, carry):
        off = pl.multiple_of(base + c * K, 8)
        pltpu.sync_copy(row_hbm.at[pl.ds(off, K)], idx_r)
        pltpu.sync_copy(col_hbm.at[pl.ds(off, K)], idx_c)
        pltpu.async_copy(s_hbm.at[idx_r], rows_v, sem).wait()
        pltpu.sync_copy(rows_v, acc_sh.at[idx_c], add=True)
        return carry
    lax.fori_loop(0, NCHUNK, step, 0)

    plsc.subcore_barrier()
    pltpu.sync_copy(acc_sh.at[pl.ds(sid * STRIP, STRIP)],
                    out_hbm.at[cid, pl.ds(sid * STRIP, STRIP)])


_agg_call = pl.kernel(
    _agg_body,
    out_type=jax.ShapeDtypeStruct((NC, NPAD, HID_DIM), jnp.float32),
    mesh=_SC_MESH,
    scratch_types=[
        pltpu.VMEM((K,), jnp.int32),
        pltpu.VMEM((K,), jnp.int32),
        pltpu.VMEM((K, HID_DIM), jnp.float32),
        pltpu.VMEM_SHARED((NPAD, HID_DIM), jnp.float32),
        pltpu.SemaphoreType.DMA,
    ],
    compiler_params=pltpu.CompilerParams(use_tc_tiling_on_sc=False),
)


def _tc1_body(x_ref, w_ref, deg_ref, s1_ref, dinv_ref):
    xw = jnp.dot(x_ref[...], w_ref[...], preferred_element_type=jnp.float32)
    p = deg_ref[...]
    deg = 2.0 + p[:, 0:1] + p[:, 1:2]
    dinv = lax.rsqrt(deg)
    s1_ref[...] = xw * dinv
    dinv_ref[...] = dinv


def _tc2_body(agg_ref, s1_ref, dinv_ref, b_ref, w_ref, s2_ref):
    p = agg_ref[...]
    dinv = dinv_ref[...]
    h = dinv * (p[0] + p[1]) + 2.0 * dinv * s1_ref[...] + b_ref[...]
    h = jnp.maximum(h, 0.0)
    s2_ref[...] = jnp.dot(h, w_ref[...],
                          preferred_element_type=jnp.float32) * dinv


def _tc3_body(agg_ref, s2_ref, dinv_ref, b_ref, out_ref):
    p = agg_ref[...]
    dinv = dinv_ref[...]
    logits = dinv * (p[0] + p[1]) + 2.0 * dinv * s2_ref[...] + b_ref[...]
    m = jnp.max(logits, axis=1, keepdims=True)
    e = jnp.exp(logits - m)
    out_ref[...] = (logits - m) - jnp.log(jnp.sum(e, axis=1, keepdims=True))


def kernel(x, edge_index, W1, b1, W2, b2):
    row = edge_index[0]
    col = edge_index[1]

    deg_parts = _hist_call(col)
    degT = deg_parts.T

    s1, dinv = pl.pallas_call(
        _tc1_body,
        grid=(GRID,),
        in_specs=[
            pl.BlockSpec((ROW_BLK, IN_DIM), lambda i: (i, 0)),
            pl.BlockSpec((IN_DIM, HID_DIM), lambda i: (0, 0)),
            pl.BlockSpec((ROW_BLK, 2), lambda i: (i, 0)),
        ],
        out_specs=[
            pl.BlockSpec((ROW_BLK, HID_DIM), lambda i: (i, 0)),
            pl.BlockSpec((ROW_BLK, 1), lambda i: (i, 0)),
        ],
        out_shape=[
            jax.ShapeDtypeStruct((NPAD, HID_DIM), jnp.float32),
            jax.ShapeDtypeStruct((NPAD, 1), jnp.float32),
        ],
    )(x, W1, degT)

    agg1 = _agg_call(s1, row, col)

    s2 = pl.pallas_call(
        _tc2_body,
        grid=(GRID,),
        in_specs=[
            pl.BlockSpec((NC, ROW_BLK, HID_DIM), lambda i: (0, i, 0)),
            pl.BlockSpec((ROW_BLK, HID_DIM), lambda i: (i, 0)),
            pl.BlockSpec((ROW_BLK, 1), lambda i: (i, 0)),
            pl.BlockSpec((1, HID_DIM), lambda i: (0, 0)),
            pl.BlockSpec((HID_DIM, OUT_DIM), lambda i: (0, 0)),
        ],
        out_specs=pl.BlockSpec((ROW_BLK, OUT_DIM), lambda i: (i, 0)),
        out_shape=jax.ShapeDtypeStruct((NPAD, OUT_DIM), jnp.float32),
    )(agg1, s1, dinv, b1.reshape(1, HID_DIM), W2)

    agg2 = _agg_call(s2, row, col)

    out = pl.pallas_call(
        _tc3_body,
        grid=(GRID,),
        in_specs=[
            pl.BlockSpec((NC, ROW_BLK, OUT_DIM), lambda i: (0, i, 0)),
            pl.BlockSpec((ROW_BLK, OUT_DIM), lambda i: (i, 0)),
            pl.BlockSpec((ROW_BLK, 1), lambda i: (i, 0)),
            pl.BlockSpec((1, OUT_DIM), lambda i: (0, 0)),
        ],
        out_specs=pl.BlockSpec((ROW_BLK, OUT_DIM), lambda i: (i, 0)),
        out_shape=jax.ShapeDtypeStruct((N_NODES, OUT_DIM), jnp.float32),
    )(agg2, s2, dinv, b2.reshape(1, OUT_DIM))

    return out


# speedup vs baseline: 25.1232x; 1.7231x over previous
import jax
import jax.numpy as jnp
from jax import lax
from jax.experimental import pallas as pl
from jax.experimental.pallas import tpu as pltpu
from jax.experimental.pallas import tpu_sc as plsc

N_NODES = 10000
N_EDGES = 320000
IN_DIM = 128
HID_DIM = 64
OUT_DIM = 64

NC = 2
NS = 16
NW = NC * NS
NPAD = 10240
STRIP = NPAD // NS
PAD_NODE = 10200

K = 112
NCHUNK = 90
EPT_PAD = K * NCHUNK
E_PAD = NW * EPT_PAD

ROW_BLK = 512
GRID = NPAD // ROW_BLK

_SC_MESH = plsc.VectorSubcoreMesh(
    core_axis_name="c", subcore_axis_name="s", num_cores=NC, num_subcores=NS)


def _hist_body(colr_hbm, out_hbm, cbuf, ones_v, zb_v, deg_sh, s0, s1):
    cid = lax.axis_index("c")
    sid = lax.axis_index("s")
    tid = cid * NS + sid

    def zb(j, c):
        zb_v[pl.ds(j * 16, 16)] = jnp.zeros((16,), jnp.float32)
        return c
    lax.fori_loop(0, STRIP // 16, zb, 0)

    def ob(j, c):
        ones_v[pl.ds(j * 16, 16)] = jnp.ones((16,), jnp.float32)
        return c
    lax.fori_loop(0, K // 16, ob, 0)

    pltpu.sync_copy(colr_hbm.at[tid], cbuf)
    pltpu.sync_copy(zb_v, deg_sh.at[pl.ds(sid * STRIP, STRIP)])
    plsc.subcore_barrier()

    def drain(sem):
        pltpu.make_async_copy(colr_hbm.at[tid, 0], cbuf.at[0], sem).wait()

    def step(i, carry):
        a = 2 * i

        @pl.when(i > 0)
        def _():
            drain(s0)
        pltpu.async_copy(ones_v, deg_sh.at[cbuf.at[a]], s0, add=True)

        @pl.when(i > 0)
        def _():
            drain(s1)
        pltpu.async_copy(ones_v, deg_sh.at[cbuf.at[a + 1]], s1, add=True)
        return carry
    lax.fori_loop(0, NCHUNK // 2, step, 0)
    drain(s0)
    drain(s1)

    plsc.subcore_barrier()
    pltpu.sync_copy(deg_sh.at[pl.ds(sid * STRIP, STRIP)],
                    out_hbm.at[cid, pl.ds(sid * STRIP, STRIP)])


_hist_call = pl.kernel(
    _hist_body,
    out_type=jax.ShapeDtypeStruct((NC, NPAD), jnp.float32),
    mesh=_SC_MESH,
    scratch_types=[
        pltpu.VMEM((NCHUNK, K), jnp.int32),
        pltpu.VMEM((K,), jnp.float32),
        pltpu.VMEM((STRIP,), jnp.float32),
        pltpu.VMEM_SHARED((NPAD,), jnp.float32),
        pltpu.SemaphoreType.DMA,
        pltpu.SemaphoreType.DMA,
    ],
    compiler_params=pltpu.CompilerParams(use_tc_tiling_on_sc=False),
)


def _agg_body(s_hbm, rowr_hbm, colr_hbm, out_hbm, rbuf, cbuf, r0, r1, zb_v,
              acc_sh, g, s0, s1):
    cid = lax.axis_index("c")
    sid = lax.axis_index("s")
    tid = cid * NS + sid

    def zr(r, _):
        def zj(j, c):
            zb_v[r, pl.ds(j * 16, 16)] = jnp.zeros((16,), jnp.float32)
            return c
        return lax.fori_loop(0, HID_DIM // 16, zj, _)
    lax.fori_loop(0, 80, zr, 0)

    def zs(k, c):
        pltpu.sync_copy(zb_v, acc_sh.at[pl.ds(sid * STRIP + k * 80, 80)])
        return c
    lax.fori_loop(0, STRIP // 80, zs, 0)

    pltpu.sync_copy(rowr_hbm.at[tid], rbuf)
    pltpu.sync_copy(colr_hbm.at[tid], cbuf)
    plsc.subcore_barrier()

    def gather(c, buf, sem):
        pltpu.async_copy(s_hbm.at[rbuf.at[c]], buf, sem)

    def gwait(c, buf, sem):
        pltpu.make_async_copy(s_hbm.at[rbuf.at[c]], buf, sem).wait()

    def scat(c, buf, sem):
        pltpu.async_copy(buf, acc_sh.at[cbuf.at[c]], sem, add=True)

    def sdrain(buf, sem):
        pltpu.make_async_copy(s_hbm.at[pl.ds(0, K)], buf, sem).wait()

    gather(0, r0, g)

    def step(i, carry):
        a = 2 * i
        b = a + 1
        gwait(a, r0, g)

        @pl.when(i > 0)
        def _():
            sdrain(r1, s1)
        gather(b, r1, g)
        scat(a, r0, s0)
        gwait(b, r1, g)
        sdrain(r0, s0)

        @pl.when(a + 2 < NCHUNK)
        def _():
            gather(a + 2, r0, g)
        scat(b, r1, s1)
        return carry
    lax.fori_loop(0, NCHUNK // 2, step, 0)
    sdrain(r1, s1)

    plsc.subcore_barrier()
    pltpu.sync_copy(acc_sh.at[pl.ds(sid * STRIP, STRIP)],
                    out_hbm.at[cid, pl.ds(sid * STRIP, STRIP)])


_agg_call = pl.kernel(
    _agg_body,
    out_type=jax.ShapeDtypeStruct((NC, NPAD, HID_DIM), jnp.float32),
    mesh=_SC_MESH,
    scratch_types=[
        pltpu.VMEM((NCHUNK, K), jnp.int32),
        pltpu.VMEM((NCHUNK, K), jnp.int32),
        pltpu.VMEM((K, HID_DIM), jnp.float32),
        pltpu.VMEM((K, HID_DIM), jnp.float32),
        pltpu.VMEM((80, HID_DIM), jnp.float32),
        pltpu.VMEM_SHARED((NPAD, HID_DIM), jnp.float32),
        pltpu.SemaphoreType.DMA,
        pltpu.SemaphoreType.DMA,
        pltpu.SemaphoreType.DMA,
    ],
    compiler_params=pltpu.CompilerParams(use_tc_tiling_on_sc=False),
)


def _tc1_body(x_ref, w_ref, deg_ref, s1_ref, dinv_ref):
    xw = jnp.dot(x_ref[...], w_ref[...], preferred_element_type=jnp.float32)
    p = deg_ref[...]
    deg = 2.0 + p[:, 0:1] + p[:, 1:2]
    dinv = lax.rsqrt(deg)
    s1_ref[...] = xw * dinv
    dinv_ref[...] = dinv


def _tc2_body(agg_ref, s1_ref, dinv_ref, b_ref, w_ref, s2_ref):
    p = agg_ref[...]
    dinv = dinv_ref[...]
    h = dinv * (p[0] + p[1]) + 2.0 * dinv * s1_ref[...] + b_ref[...]
    h = jnp.maximum(h, 0.0)
    s2_ref[...] = jnp.dot(h, w_ref[...],
                          preferred_element_type=jnp.float32) * dinv


def _tc3_body(agg_ref, s2_ref, dinv_ref, b_ref, out_ref):
    p = agg_ref[...]
    dinv = dinv_ref[...]
    logits = dinv * (p[0] + p[1]) + 2.0 * dinv * s2_ref[...] + b_ref[...]
    m = jnp.max(logits, axis=1, keepdims=True)
    e = jnp.exp(logits - m)
    out_ref[...] = (logits - m) - jnp.log(jnp.sum(e, axis=1, keepdims=True))


def kernel(x, edge_index, W1, b1, W2, b2):
    ei = edge_index.astype(jnp.int32)
    pad = jnp.full((2, E_PAD - N_EDGES), PAD_NODE, jnp.int32)
    ei_pad = jnp.concatenate([ei, pad], axis=1)
    row_r = ei_pad[0].reshape(NW, NCHUNK, K)
    col_r = ei_pad[1].reshape(NW, NCHUNK, K)

    deg_parts = _hist_call(col_r)
    degT = deg_parts.T

    s1, dinv = pl.pallas_call(
        _tc1_body,
        grid=(GRID,),
        in_specs=[
            pl.BlockSpec((ROW_BLK, IN_DIM), lambda i: (i, 0)),
            pl.BlockSpec((IN_DIM, HID_DIM), lambda i: (0, 0)),
            pl.BlockSpec((ROW_BLK, 2), lambda i: (i, 0)),
        ],
        out_specs=[
            pl.BlockSpec((ROW_BLK, HID_DIM), lambda i: (i, 0)),
            pl.BlockSpec((ROW_BLK, 1), lambda i: (i, 0)),
        ],
        out_shape=[
            jax.ShapeDtypeStruct((NPAD, HID_DIM), jnp.float32),
            jax.ShapeDtypeStruct((NPAD, 1), jnp.float32),
        ],
    )(x, W1, degT)

    agg1 = _agg_call(s1, row_r, col_r)

    s2 = pl.pallas_call(
        _tc2_body,
        grid=(GRID,),
        in_specs=[
            pl.BlockSpec((NC, ROW_BLK, HID_DIM), lambda i: (0, i, 0)),
            pl.BlockSpec((ROW_BLK, HID_DIM), lambda i: (i, 0)),
            pl.BlockSpec((ROW_BLK, 1), lambda i: (i, 0)),
            pl.BlockSpec((1, HID_DIM), lambda i: (0, 0)),
            pl.BlockSpec((HID_DIM, OUT_DIM), lambda i: (0, 0)),
        ],
        out_specs=pl.BlockSpec((ROW_BLK, OUT_DIM), lambda i: (i, 0)),
        out_shape=jax.ShapeDtypeStruct((NPAD, OUT_DIM), jnp.float32),
    )(agg1, s1, dinv, b1.reshape(1, HID_DIM), W2)

    agg2 = _agg_call(s2, row_r, col_r)

    out = pl.pallas_call(
        _tc3_body,
        grid=(GRID,),
        in_specs=[
            pl.BlockSpec((NC, ROW_BLK, OUT_DIM), lambda i: (0, i, 0)),
            pl.BlockSpec((ROW_BLK, OUT_DIM), lambda i: (i, 0)),
            pl.BlockSpec((ROW_BLK, 1), lambda i: (i, 0)),
            pl.BlockSpec((1, OUT_DIM), lambda i: (0, 0)),
        ],
        out_specs=pl.BlockSpec((ROW_BLK, OUT_DIM), lambda i: (i, 0)),
        out_shape=jax.ShapeDtypeStruct((N_NODES, OUT_DIM), jnp.float32),
    )(agg2, s2, dinv, b2.reshape(1, OUT_DIM))

    return out


# speedup vs baseline: 33.1308x; 1.3187x over previous
import jax
import jax.numpy as jnp
from jax import lax
from jax.experimental import pallas as pl
from jax.experimental.pallas import tpu as pltpu
from jax.experimental.pallas import tpu_sc as plsc

N_NODES = 10000
N_EDGES = 320000
IN_DIM = 128
HID_DIM = 64
OUT_DIM = 64

NC = 2
NS = 16
NW = NC * NS
NPAD = 10240
STRIP = NPAD // NS
K = 128
NCHUNK = 80
EPT = N_EDGES // NW
EPT_PAD = K * NCHUNK
PAD_PER_TILE = EPT_PAD - EPT

ROW_BLK = 512
GRID = NPAD // ROW_BLK

_SC_MESH = plsc.VectorSubcoreMesh(
    core_axis_name="c", subcore_axis_name="s", num_cores=NC, num_subcores=NS)


def _hist_body(colr_hbm, out_hbm, cbuf, ones_v, zb_v, deg_sh, s0, s1):
    cid = lax.axis_index("c")
    sid = lax.axis_index("s")
    tid = cid * NS + sid

    def zb(j, c):
        zb_v[pl.ds(j * 16, 16)] = jnp.zeros((16,), jnp.float32)
        return c
    lax.fori_loop(0, STRIP // 16, zb, 0)

    def ob(j, c):
        ones_v[pl.ds(j * 16, 16)] = jnp.ones((16,), jnp.float32)
        return c
    lax.fori_loop(0, K // 16, ob, 0)

    pltpu.sync_copy(colr_hbm.at[tid], cbuf)
    pltpu.sync_copy(zb_v, deg_sh.at[pl.ds(sid * STRIP, STRIP)])
    plsc.subcore_barrier()

    def drain(sem):
        pltpu.make_async_copy(colr_hbm.at[tid, 0], cbuf.at[0], sem).wait()

    def step(i, carry):
        a = 2 * i

        @pl.when(i > 0)
        def _():
            drain(s0)
        pltpu.async_copy(ones_v, deg_sh.at[cbuf.at[a]], s0, add=True)

        @pl.when(i > 0)
        def _():
            drain(s1)
        pltpu.async_copy(ones_v, deg_sh.at[cbuf.at[a + 1]], s1, add=True)
        return carry
    lax.fori_loop(0, NCHUNK // 2, step, 0)
    drain(s0)
    drain(s1)

    plsc.subcore_barrier()
    pltpu.sync_copy(deg_sh.at[pl.ds(sid * STRIP, STRIP)],
                    out_hbm.at[cid, pl.ds(sid * STRIP, STRIP)])


_hist_call = pl.kernel(
    _hist_body,
    out_type=jax.ShapeDtypeStruct((NC, NPAD), jnp.float32),
    mesh=_SC_MESH,
    scratch_types=[
        pltpu.VMEM((NCHUNK, K), jnp.int32),
        pltpu.VMEM((K,), jnp.float32),
        pltpu.VMEM((STRIP,), jnp.float32),
        pltpu.VMEM_SHARED((NPAD,), jnp.float32),
        pltpu.SemaphoreType.DMA,
        pltpu.SemaphoreType.DMA,
    ],
    compiler_params=pltpu.CompilerParams(use_tc_tiling_on_sc=False),
)


def _agg_body(s_hbm, rowr_hbm, colr_hbm, out_hbm, rbuf, cbuf, r0, r1, zb_v,
              acc_sh, g, s0, s1):
    cid = lax.axis_index("c")
    sid = lax.axis_index("s")
    tid = cid * NS + sid

    def zr(r, _):
        def zj(j, c):
            zb_v[r, pl.ds(j * 16, 16)] = jnp.zeros((16,), jnp.float32)
            return c
        return lax.fori_loop(0, HID_DIM // 16, zj, _)
    lax.fori_loop(0, 80, zr, 0)

    def zs(k, c):
        pltpu.sync_copy(zb_v, acc_sh.at[pl.ds(sid * STRIP + k * 80, 80)])
        return c
    lax.fori_loop(0, STRIP // 80, zs, 0)

    pltpu.sync_copy(rowr_hbm.at[tid], rbuf)
    pltpu.sync_copy(colr_hbm.at[tid], cbuf)
    plsc.subcore_barrier()

    def gather(c, buf, sem):
        pltpu.async_copy(s_hbm.at[rbuf.at[c]], buf, sem)

    def gwait(c, buf, sem):
        pltpu.make_async_copy(s_hbm.at[rbuf.at[c]], buf, sem).wait()

    def scat(c, buf, sem):
        pltpu.async_copy(buf, acc_sh.at[cbuf.at[c]], sem, add=True)

    def sdrain(buf, sem):
        pltpu.make_async_copy(s_hbm.at[pl.ds(0, K)], buf, sem).wait()

    gather(0, r0, g)

    def step(i, carry):
        a = 2 * i
        b = a + 1
        gwait(a, r0, g)

        @pl.when(i > 0)
        def _():
            sdrain(r1, s1)
        gather(b, r1, g)
        scat(a, r0, s0)
        gwait(b, r1, g)
        sdrain(r0, s0)

        @pl.when(a + 2 < NCHUNK)
        def _():
            gather(a + 2, r0, g)
        scat(b, r1, s1)
        return carry
    lax.fori_loop(0, NCHUNK // 2, step, 0)
    sdrain(r1, s1)

    plsc.subcore_barrier()
    pltpu.sync_copy(acc_sh.at[pl.ds(sid * STRIP, STRIP)],
                    out_hbm.at[cid, pl.ds(sid * STRIP, STRIP)])


_agg_call = pl.kernel(
    _agg_body,
    out_type=jax.ShapeDtypeStruct((NC, NPAD, HID_DIM), jnp.float32),
    mesh=_SC_MESH,
    scratch_types=[
        pltpu.VMEM((NCHUNK, K), jnp.int32),
        pltpu.VMEM((NCHUNK, K), jnp.int32),
        pltpu.VMEM((K, HID_DIM), jnp.float32),
        pltpu.VMEM((K, HID_DIM), jnp.float32),
        pltpu.VMEM((80, HID_DIM), jnp.float32),
        pltpu.VMEM_SHARED((NPAD, HID_DIM), jnp.float32),
        pltpu.SemaphoreType.DMA,
        pltpu.SemaphoreType.DMA,
        pltpu.SemaphoreType.DMA,
    ],
    compiler_params=pltpu.CompilerParams(use_tc_tiling_on_sc=False),
)


def _tc1_body(x_ref, w_ref, deg_ref, s1_ref, dinv_ref):
    xw = jnp.dot(x_ref[...], w_ref[...], preferred_element_type=jnp.float32)
    p = deg_ref[...]
    deg = 2.0 + p[:, 0:1] + p[:, 1:2]
    dinv = lax.rsqrt(deg)
    s1_ref[...] = xw * dinv
    dinv_ref[...] = dinv


def _tc2_body(agg_ref, s1_ref, dinv_ref, b_ref, w_ref, s2_ref):
    p = agg_ref[...]
    dinv = dinv_ref[...]
    h = dinv * (p[0] + p[1]) + 2.0 * dinv * s1_ref[...] + b_ref[...]
    h = jnp.maximum(h, 0.0)
    s2_ref[...] = jnp.dot(h, w_ref[...],
                          preferred_element_type=jnp.float32) * dinv


def _tc3_body(agg_ref, s2_ref, dinv_ref, b_ref, out_ref):
    p = agg_ref[...]
    dinv = dinv_ref[...]
    logits = dinv * (p[0] + p[1]) + 2.0 * dinv * s2_ref[...] + b_ref[...]
    m = jnp.max(logits, axis=1, keepdims=True)
    e = jnp.exp(logits - m)
    out_ref[...] = (logits - m) - jnp.log(jnp.sum(e, axis=1, keepdims=True))


def kernel(x, edge_index, W1, b1, W2, b2):
    ei = edge_index.astype(jnp.int32).reshape(2, NW, EPT)
    pad = jnp.broadcast_to(
        N_NODES + jnp.arange(PAD_PER_TILE, dtype=jnp.int32),
        (2, NW, PAD_PER_TILE))
    ei_pad = jnp.concatenate([ei, pad], axis=2)
    row_r = ei_pad[0].reshape(NW, NCHUNK, K)
    col_r = ei_pad[1].reshape(NW, NCHUNK, K)

    deg_parts = _hist_call(col_r)
    degT = deg_parts.T

    s1, dinv = pl.pallas_call(
        _tc1_body,
        grid=(GRID,),
        in_specs=[
            pl.BlockSpec((ROW_BLK, IN_DIM), lambda i: (i, 0)),
            pl.BlockSpec((IN_DIM, HID_DIM), lambda i: (0, 0)),
            pl.BlockSpec((ROW_BLK, 2), lambda i: (i, 0)),
        ],
        out_specs=[
            pl.BlockSpec((ROW_BLK, HID_DIM), lambda i: (i, 0)),
            pl.BlockSpec((ROW_BLK, 1), lambda i: (i, 0)),
        ],
        out_shape=[
            jax.ShapeDtypeStruct((NPAD, HID_DIM), jnp.float32),
            jax.ShapeDtypeStruct((NPAD, 1), jnp.float32),
        ],
    )(x, W1, degT)

    agg1 = _agg_call(s1, row_r, col_r)

    s2 = pl.pallas_call(
        _tc2_body,
        grid=(GRID,),
        in_specs=[
            pl.BlockSpec((NC, ROW_BLK, HID_DIM), lambda i: (0, i, 0)),
            pl.BlockSpec((ROW_BLK, HID_DIM), lambda i: (i, 0)),
            pl.BlockSpec((ROW_BLK, 1), lambda i: (i, 0)),
            pl.BlockSpec((1, HID_DIM), lambda i: (0, 0)),
            pl.BlockSpec((HID_DIM, OUT_DIM), lambda i: (0, 0)),
        ],
        out_specs=pl.BlockSpec((ROW_BLK, OUT_DIM), lambda i: (i, 0)),
        out_shape=jax.ShapeDtypeStruct((NPAD, OUT_DIM), jnp.float32),
    )(agg1, s1, dinv, b1.reshape(1, HID_DIM), W2)

    agg2 = _agg_call(s2, row_r, col_r)

    out = pl.pallas_call(
        _tc3_body,
        grid=(GRID,),
        in_specs=[
            pl.BlockSpec((NC, ROW_BLK, OUT_DIM), lambda i: (0, i, 0)),
            pl.BlockSpec((ROW_BLK, OUT_DIM), lambda i: (i, 0)),
            pl.BlockSpec((ROW_BLK, 1), lambda i: (i, 0)),
            pl.BlockSpec((1, OUT_DIM), lambda i: (0, 0)),
        ],
        out_specs=pl.BlockSpec((ROW_BLK, OUT_DIM), lambda i: (i, 0)),
        out_shape=jax.ShapeDtypeStruct((N_NODES, OUT_DIM), jnp.float32),
    )(agg2, s2, dinv, b2.reshape(1, OUT_DIM))

    return out


# speedup vs baseline: 38.7988x; 1.1711x over previous
import jax
import jax.numpy as jnp
from jax import lax
from jax.experimental import pallas as pl
from jax.experimental.pallas import tpu as pltpu
from jax.experimental.pallas import tpu_sc as plsc

N_NODES = 10000
N_EDGES = 320000
IN_DIM = 128
HID_DIM = 64
OUT_DIM = 64

NC = 2
NS = 16
NW = NC * NS
NPAD = 10240
STRIP = NPAD // NS
K = 128
NCHUNK = 81
NITER = NCHUNK // 3
EPT = N_EDGES // NW
EPT_PAD = K * NCHUNK
PAD_PER_TILE = EPT_PAD - EPT

ROW_BLK = 512
GRID = NPAD // ROW_BLK

_SC_MESH = plsc.VectorSubcoreMesh(
    core_axis_name="c", subcore_axis_name="s", num_cores=NC, num_subcores=NS)


def _hist_body(colr_hbm, out_hbm, cbuf, ones_v, zb_v, deg_sh, s0, s1, s2):
    cid = lax.axis_index("c")
    sid = lax.axis_index("s")
    tid = cid * NS + sid

    def zb(j, c):
        zb_v[pl.ds(j * 16, 16)] = jnp.zeros((16,), jnp.float32)
        return c
    lax.fori_loop(0, STRIP // 16, zb, 0)

    def ob(j, c):
        ones_v[pl.ds(j * 16, 16)] = jnp.ones((16,), jnp.float32)
        return c
    lax.fori_loop(0, K // 16, ob, 0)

    pltpu.sync_copy(colr_hbm.at[tid], cbuf)
    pltpu.sync_copy(zb_v, deg_sh.at[pl.ds(sid * STRIP, STRIP)])
    plsc.subcore_barrier()

    def drain(sem):
        pltpu.make_async_copy(colr_hbm.at[tid, 0], cbuf.at[0], sem).wait()

    def slot(i, j, sem):
        @pl.when(i > 0)
        def _():
            drain(sem)
        pltpu.async_copy(ones_v, deg_sh.at[cbuf.at[3 * i + j]], sem, add=True)

    def step(i, carry):
        slot(i, 0, s0)
        slot(i, 1, s1)
        slot(i, 2, s2)
        return carry
    lax.fori_loop(0, NITER, step, 0)
    drain(s0)
    drain(s1)
    drain(s2)

    plsc.subcore_barrier()
    pltpu.sync_copy(deg_sh.at[pl.ds(sid * STRIP, STRIP)],
                    out_hbm.at[cid, pl.ds(sid * STRIP, STRIP)])


_hist_call = pl.kernel(
    _hist_body,
    out_type=jax.ShapeDtypeStruct((NC, NPAD), jnp.float32),
    mesh=_SC_MESH,
    scratch_types=[
        pltpu.VMEM((NCHUNK, K), jnp.int32),
        pltpu.VMEM((K,), jnp.float32),
        pltpu.VMEM((STRIP,), jnp.float32),
        pltpu.VMEM_SHARED((NPAD,), jnp.float32),
        pltpu.SemaphoreType.DMA,
        pltpu.SemaphoreType.DMA,
        pltpu.SemaphoreType.DMA,
    ],
    compiler_params=pltpu.CompilerParams(use_tc_tiling_on_sc=False),
)


def _agg_body(s_hbm, rowr_hbm, colr_hbm, out_hbm, rbuf, cbuf, r0, r1, r2,
              zb_v, acc_sh, g0, g1, g2, s0, s1, s2):
    cid = lax.axis_index("c")
    sid = lax.axis_index("s")
    tid = cid * NS + sid

    def zr(r, _):
        def zj(j, c):
            zb_v[r, pl.ds(j * 16, 16)] = jnp.zeros((16,), jnp.float32)
            return c
        return lax.fori_loop(0, HID_DIM // 16, zj, _)
    lax.fori_loop(0, 80, zr, 0)

    def zs(k, c):
        pltpu.sync_copy(zb_v, acc_sh.at[pl.ds(sid * STRIP + k * 80, 80)])
        return c
    lax.fori_loop(0, STRIP // 80, zs, 0)

    pltpu.sync_copy(rowr_hbm.at[tid], rbuf)
    pltpu.sync_copy(colr_hbm.at[tid], cbuf)
    plsc.subcore_barrier()

    def gather(c, buf, sem):
        pltpu.async_copy(s_hbm.at[rbuf.at[c]], buf, sem)

    def gwait(c, buf, sem):
        pltpu.make_async_copy(s_hbm.at[rbuf.at[c]], buf, sem).wait()

    def scat(c, buf, sem):
        pltpu.async_copy(buf, acc_sh.at[cbuf.at[c]], sem, add=True)

    def sdrain(buf, sem):
        pltpu.make_async_copy(s_hbm.at[pl.ds(0, K)], buf, sem).wait()

    gather(0, r0, g0)
    gather(1, r1, g1)
    gather(2, r2, g2)

    def refill(i, j, buf, gsem, ssem):
        sdrain(buf, ssem)
        gather(3 * (i + 1) + j, buf, gsem)

    def step(i, carry):
        a = 3 * i
        gwait(a, r0, g0)
        scat(a, r0, s0)
        gwait(a + 1, r1, g1)
        scat(a + 1, r1, s1)
        gwait(a + 2, r2, g2)
        scat(a + 2, r2, s2)

        @pl.when(i + 1 < NITER)
        def _():
            refill(i, 0, r0, g0, s0)
            refill(i, 1, r1, g1, s1)
            refill(i, 2, r2, g2, s2)
        return carry
    lax.fori_loop(0, NITER, step, 0)
    sdrain(r0, s0)
    sdrain(r1, s1)
    sdrain(r2, s2)

    plsc.subcore_barrier()
    pltpu.sync_copy(acc_sh.at[pl.ds(sid * STRIP, STRIP)],
                    out_hbm.at[cid, pl.ds(sid * STRIP, STRIP)])


_agg_call = pl.kernel(
    _agg_body,
    out_type=jax.ShapeDtypeStruct((NC, NPAD, HID_DIM), jnp.float32),
    mesh=_SC_MESH,
    scratch_types=[
        pltpu.VMEM((NCHUNK, K), jnp.int32),
        pltpu.VMEM((NCHUNK, K), jnp.int32),
        pltpu.VMEM((K, HID_DIM), jnp.float32),
        pltpu.VMEM((K, HID_DIM), jnp.float32),
        pltpu.VMEM((K, HID_DIM), jnp.float32),
        pltpu.VMEM((80, HID_DIM), jnp.float32),
        pltpu.VMEM_SHARED((NPAD, HID_DIM), jnp.float32),
        pltpu.SemaphoreType.DMA,
        pltpu.SemaphoreType.DMA,
        pltpu.SemaphoreType.DMA,
        pltpu.SemaphoreType.DMA,
        pltpu.SemaphoreType.DMA,
        pltpu.SemaphoreType.DMA,
    ],
    compiler_params=pltpu.CompilerParams(use_tc_tiling_on_sc=False),
)


def _tc1_body(x_ref, w_ref, deg_ref, s1_ref, dinv_ref):
    xw = jnp.dot(x_ref[...], w_ref[...], preferred_element_type=jnp.float32)
    p = deg_ref[...]
    deg = 2.0 + p[:, 0:1] + p[:, 1:2]
    dinv = lax.rsqrt(deg)
    s1_ref[...] = xw * dinv
    dinv_ref[...] = dinv


def _tc2_body(agg_ref, s1_ref, dinv_ref, b_ref, w_ref, s2_ref):
    p = agg_ref[...]
    dinv = dinv_ref[...]
    h = dinv * (p[0] + p[1]) + 2.0 * dinv * s1_ref[...] + b_ref[...]
    h = jnp.maximum(h, 0.0)
    s2_ref[...] = jnp.dot(h, w_ref[...],
                          preferred_element_type=jnp.float32) * dinv


def _tc3_body(agg_ref, s2_ref, dinv_ref, b_ref, out_ref):
    p = agg_ref[...]
    dinv = dinv_ref[...]
    logits = dinv * (p[0] + p[1]) + 2.0 * dinv * s2_ref[...] + b_ref[...]
    m = jnp.max(logits, axis=1, keepdims=True)
    e = jnp.exp(logits - m)
    out_ref[...] = (logits - m) - jnp.log(jnp.sum(e, axis=1, keepdims=True))


def kernel(x, edge_index, W1, b1, W2, b2):
    ei = edge_index.astype(jnp.int32).reshape(2, NW, EPT)
    pad = jnp.broadcast_to(
        N_NODES + jnp.arange(PAD_PER_TILE, dtype=jnp.int32) % (NPAD - N_NODES),
        (2, NW, PAD_PER_TILE))
    ei_pad = jnp.concatenate([ei, pad], axis=2)
    row_r = ei_pad[0].reshape(NW, NCHUNK, K)
    col_r = ei_pad[1].reshape(NW, NCHUNK, K)

    deg_parts = _hist_call(col_r)
    degT = deg_parts.T

    s1, dinv = pl.pallas_call(
        _tc1_body,
        grid=(GRID,),
        in_specs=[
            pl.BlockSpec((ROW_BLK, IN_DIM), lambda i: (i, 0)),
            pl.BlockSpec((IN_DIM, HID_DIM), lambda i: (0, 0)),
            pl.BlockSpec((ROW_BLK, 2), lambda i: (i, 0)),
        ],
        out_specs=[
            pl.BlockSpec((ROW_BLK, HID_DIM), lambda i: (i, 0)),
            pl.BlockSpec((ROW_BLK, 1), lambda i: (i, 0)),
        ],
        out_shape=[
            jax.ShapeDtypeStruct((NPAD, HID_DIM), jnp.float32),
            jax.ShapeDtypeStruct((NPAD, 1), jnp.float32),
        ],
    )(x, W1, degT)

    agg1 = _agg_call(s1, row_r, col_r)

    s2 = pl.pallas_call(
        _tc2_body,
        grid=(GRID,),
        in_specs=[
            pl.BlockSpec((NC, ROW_BLK, HID_DIM), lambda i: (0, i, 0)),
            pl.BlockSpec((ROW_BLK, HID_DIM), lambda i: (i, 0)),
            pl.BlockSpec((ROW_BLK, 1), lambda i: (i, 0)),
            pl.BlockSpec((1, HID_DIM), lambda i: (0, 0)),
            pl.BlockSpec((HID_DIM, OUT_DIM), lambda i: (0, 0)),
        ],
        out_specs=pl.BlockSpec((ROW_BLK, OUT_DIM), lambda i: (i, 0)),
        out_shape=jax.ShapeDtypeStruct((NPAD, OUT_DIM), jnp.float32),
    )(agg1, s1, dinv, b1.reshape(1, HID_DIM), W2)

    agg2 = _agg_call(s2, row_r, col_r)

    out = pl.pallas_call(
        _tc3_body,
        grid=(GRID,),
        in_specs=[
            pl.BlockSpec((NC, ROW_BLK, OUT_DIM), lambda i: (0, i, 0)),
            pl.BlockSpec((ROW_BLK, OUT_DIM), lambda i: (i, 0)),
            pl.BlockSpec((ROW_BLK, 1), lambda i: (i, 0)),
            pl.BlockSpec((1, OUT_DIM), lambda i: (0, 0)),
        ],
        out_specs=pl.BlockSpec((ROW_BLK, OUT_DIM), lambda i: (i, 0)),
        out_shape=jax.ShapeDtypeStruct((N_NODES, OUT_DIM), jnp.float32),
    )(agg2, s2, dinv, b2.reshape(1, OUT_DIM))

    return out


# speedup vs baseline: 43.1808x; 1.1129x over previous
import jax
import jax.numpy as jnp
from jax import lax
from jax.experimental import pallas as pl
from jax.experimental.pallas import tpu as pltpu
from jax.experimental.pallas import tpu_sc as plsc

N_NODES = 10000
N_EDGES = 320000
IN_DIM = 128
HID_DIM = 64
OUT_DIM = 64

NC = 2
NS = 16
NW = NC * NS
NPAD = 10240
STRIP = NPAD // NS
K = 128
NCHUNK = 81
NITER = NCHUNK // 3
EPT = N_EDGES // NW
EPT_PAD = K * NCHUNK
PAD_PER_TILE = EPT_PAD - EPT

ROW_BLK = 2048
GRID = NPAD // ROW_BLK

_SC_MESH = plsc.VectorSubcoreMesh(
    core_axis_name="c", subcore_axis_name="s", num_cores=NC, num_subcores=NS)


def _hist_body(colr_hbm, out_hbm, cbuf, ones_v, zb_v, deg_sh, s0, s1, s2):
    cid = lax.axis_index("c")
    sid = lax.axis_index("s")
    tid = cid * NS + sid

    def zb(j, c):
        zb_v[pl.ds(j * 16, 16)] = jnp.zeros((16,), jnp.float32)
        return c
    lax.fori_loop(0, STRIP // 16, zb, 0)

    def ob(j, c):
        ones_v[pl.ds(j * 16, 16)] = jnp.ones((16,), jnp.float32)
        return c
    lax.fori_loop(0, K // 16, ob, 0)

    pltpu.sync_copy(colr_hbm.at[tid], cbuf)
    pltpu.sync_copy(zb_v, deg_sh.at[pl.ds(sid * STRIP, STRIP)])
    plsc.subcore_barrier()

    def drain(sem):
        pltpu.make_async_copy(colr_hbm.at[tid, 0], cbuf.at[0], sem).wait()

    def slot(i, j, sem):
        @pl.when(i > 0)
        def _():
            drain(sem)
        pltpu.async_copy(ones_v, deg_sh.at[cbuf.at[3 * i + j]], sem, add=True)

    def step(i, carry):
        slot(i, 0, s0)
        slot(i, 1, s1)
        slot(i, 2, s2)
        return carry
    lax.fori_loop(0, NITER, step, 0)
    drain(s0)
    drain(s1)
    drain(s2)

    plsc.subcore_barrier()
    pltpu.sync_copy(deg_sh.at[pl.ds(sid * STRIP, STRIP)],
                    out_hbm.at[cid, pl.ds(sid * STRIP, STRIP)])


_hist_call = pl.kernel(
    _hist_body,
    out_type=jax.ShapeDtypeStruct((NC, NPAD), jnp.float32),
    mesh=_SC_MESH,
    scratch_types=[
        pltpu.VMEM((NCHUNK, K), jnp.int32),
        pltpu.VMEM((K,), jnp.float32),
        pltpu.VMEM((STRIP,), jnp.float32),
        pltpu.VMEM_SHARED((NPAD,), jnp.float32),
        pltpu.SemaphoreType.DMA,
        pltpu.SemaphoreType.DMA,
        pltpu.SemaphoreType.DMA,
    ],
    compiler_params=pltpu.CompilerParams(use_tc_tiling_on_sc=False),
)


def _agg_body(s_hbm, rowr_hbm, colr_hbm, out_hbm, rbuf, cbuf, r0, r1, r2,
              zb_v, acc_sh, g0, g1, g2, s0, s1, s2):
    cid = lax.axis_index("c")
    sid = lax.axis_index("s")
    tid = cid * NS + sid

    def zr(r, _):
        def zj(j, c):
            zb_v[r, pl.ds(j * 16, 16)] = jnp.zeros((16,), jnp.float32)
            return c
        return lax.fori_loop(0, HID_DIM // 16, zj, _)
    lax.fori_loop(0, 80, zr, 0)

    def zs(k, c):
        pltpu.sync_copy(zb_v, acc_sh.at[pl.ds(sid * STRIP + k * 80, 80)])
        return c
    lax.fori_loop(0, STRIP // 80, zs, 0)

    pltpu.sync_copy(rowr_hbm.at[tid], rbuf)
    pltpu.sync_copy(colr_hbm.at[tid], cbuf)
    plsc.subcore_barrier()

    def gather(c, buf, sem):
        pltpu.async_copy(s_hbm.at[rbuf.at[c]], buf, sem)

    def gwait(c, buf, sem):
        pltpu.make_async_copy(s_hbm.at[rbuf.at[c]], buf, sem).wait()

    def scat(c, buf, sem):
        pltpu.async_copy(buf, acc_sh.at[cbuf.at[c]], sem, add=True)

    def sdrain(buf, sem):
        pltpu.make_async_copy(s_hbm.at[pl.ds(0, K)], buf, sem).wait()

    gather(0, r0, g0)
    gather(1, r1, g1)
    gather(2, r2, g2)

    def refill(i, j, buf, gsem, ssem):
        sdrain(buf, ssem)
        gather(3 * (i + 1) + j, buf, gsem)

    def step(i, carry):
        a = 3 * i
        gwait(a, r0, g0)
        scat(a, r0, s0)
        gwait(a + 1, r1, g1)
        scat(a + 1, r1, s1)
        gwait(a + 2, r2, g2)
        scat(a + 2, r2, s2)

        @pl.when(i + 1 < NITER)
        def _():
            refill(i, 0, r0, g0, s0)
            refill(i, 1, r1, g1, s1)
            refill(i, 2, r2, g2, s2)
        return carry
    lax.fori_loop(0, NITER, step, 0)
    sdrain(r0, s0)
    sdrain(r1, s1)
    sdrain(r2, s2)

    plsc.subcore_barrier()
    pltpu.sync_copy(acc_sh.at[pl.ds(sid * STRIP, STRIP)],
                    out_hbm.at[cid, pl.ds(sid * STRIP, STRIP)])


_agg_call = pl.kernel(
    _agg_body,
    out_type=jax.ShapeDtypeStruct((NC, NPAD, HID_DIM), jnp.float32),
    mesh=_SC_MESH,
    scratch_types=[
        pltpu.VMEM((NCHUNK, K), jnp.int32),
        pltpu.VMEM((NCHUNK, K), jnp.int32),
        pltpu.VMEM((K, HID_DIM), jnp.float32),
        pltpu.VMEM((K, HID_DIM), jnp.float32),
        pltpu.VMEM((K, HID_DIM), jnp.float32),
        pltpu.VMEM((80, HID_DIM), jnp.float32),
        pltpu.VMEM_SHARED((NPAD, HID_DIM), jnp.float32),
        pltpu.SemaphoreType.DMA,
        pltpu.SemaphoreType.DMA,
        pltpu.SemaphoreType.DMA,
        pltpu.SemaphoreType.DMA,
        pltpu.SemaphoreType.DMA,
        pltpu.SemaphoreType.DMA,
    ],
    compiler_params=pltpu.CompilerParams(use_tc_tiling_on_sc=False),
)


def _tc1_body(x_ref, w_ref, deg_ref, s1_ref, dinv_ref):
    xw = jnp.dot(x_ref[...], w_ref[...], preferred_element_type=jnp.float32)
    p = deg_ref[...]
    deg_row = 2.0 + p[0:1, :] + p[1:2, :]
    dinv = jnp.transpose(lax.rsqrt(deg_row), (1, 0))
    s1_ref[...] = xw * dinv
    dinv_ref[...] = dinv


def _tc2_body(agg_ref, s1_ref, dinv_ref, b_ref, w_ref, s2_ref):
    p = agg_ref[...]
    dinv = dinv_ref[...]
    h = dinv * (p[0] + p[1]) + 2.0 * dinv * s1_ref[...] + b_ref[...]
    h = jnp.maximum(h, 0.0)
    s2_ref[...] = jnp.dot(h, w_ref[...],
                          preferred_element_type=jnp.float32) * dinv


def _tc3_body(agg_ref, s2_ref, dinv_ref, b_ref, out_ref):
    p = agg_ref[...]
    dinv = dinv_ref[...]
    logits = dinv * (p[0] + p[1]) + 2.0 * dinv * s2_ref[...] + b_ref[...]
    m = jnp.max(logits, axis=1, keepdims=True)
    e = jnp.exp(logits - m)
    out_ref[...] = (logits - m) - jnp.log(jnp.sum(e, axis=1, keepdims=True))


def kernel(x, edge_index, W1, b1, W2, b2):
    ei = edge_index.astype(jnp.int32).reshape(2, NW, EPT)
    pad = jnp.broadcast_to(
        N_NODES + jnp.arange(PAD_PER_TILE, dtype=jnp.int32) % (NPAD - N_NODES),
        (2, NW, PAD_PER_TILE))
    ei_pad = jnp.concatenate([ei, pad], axis=2)
    row_r = ei_pad[0].reshape(NW, NCHUNK, K)
    col_r = ei_pad[1].reshape(NW, NCHUNK, K)

    deg_parts = _hist_call(col_r)

    s1, dinv = pl.pallas_call(
        _tc1_body,
        grid=(GRID,),
        in_specs=[
            pl.BlockSpec((ROW_BLK, IN_DIM), lambda i: (i, 0)),
            pl.BlockSpec((IN_DIM, HID_DIM), lambda i: (0, 0)),
            pl.BlockSpec((2, ROW_BLK), lambda i: (0, i)),
        ],
        out_specs=[
            pl.BlockSpec((ROW_BLK, HID_DIM), lambda i: (i, 0)),
            pl.BlockSpec((ROW_BLK, 1), lambda i: (i, 0)),
        ],
        out_shape=[
            jax.ShapeDtypeStruct((NPAD, HID_DIM), jnp.float32),
            jax.ShapeDtypeStruct((NPAD, 1), jnp.float32),
        ],
    )(x, W1, deg_parts)

    agg1 = _agg_call(s1, row_r, col_r)

    s2 = pl.pallas_call(
        _tc2_body,
        grid=(GRID,),
        in_specs=[
            pl.BlockSpec((NC, ROW_BLK, HID_DIM), lambda i: (0, i, 0)),
            pl.BlockSpec((ROW_BLK, HID_DIM), lambda i: (i, 0)),
            pl.BlockSpec((ROW_BLK, 1), lambda i: (i, 0)),
            pl.BlockSpec((1, HID_DIM), lambda i: (0, 0)),
            pl.BlockSpec((HID_DIM, OUT_DIM), lambda i: (0, 0)),
        ],
        out_specs=pl.BlockSpec((ROW_BLK, OUT_DIM), lambda i: (i, 0)),
        out_shape=jax.ShapeDtypeStruct((NPAD, OUT_DIM), jnp.float32),
    )(agg1, s1, dinv, b1.reshape(1, HID_DIM), W2)

    agg2 = _agg_call(s2, row_r, col_r)

    out = pl.pallas_call(
        _tc3_body,
        grid=(GRID,),
        in_specs=[
            pl.BlockSpec((NC, ROW_BLK, OUT_DIM), lambda i: (0, i, 0)),
            pl.BlockSpec((ROW_BLK, OUT_DIM), lambda i: (i, 0)),
            pl.BlockSpec((ROW_BLK, 1), lambda i: (i, 0)),
            pl.BlockSpec((1, OUT_DIM), lambda i: (0, 0)),
        ],
        out_specs=pl.BlockSpec((ROW_BLK, OUT_DIM), lambda i: (i, 0)),
        out_shape=jax.ShapeDtypeStruct((N_NODES, OUT_DIM), jnp.float32),
    )(agg2, s2, dinv, b2.reshape(1, OUT_DIM))

    return out
